# baseline (speedup 1.0000x reference)
import functools

import jax
import jax.numpy as jnp
from jax import lax
from jax.experimental import pallas as pl
from jax.experimental.pallas import tpu as pltpu
from jax.experimental.pallas import tpu_sc as plsc

DIM = 256
N_EMBED = 8192
M_TOKENS = 8192

BM = 1024
BN = 1024
NM = M_TOKENS // BM
NN = N_EMBED // BN


def _dist_argmin_body(x_ref, e_ref, ind_ref, dsum_ref, et_ref,
                      minv_s, mini_s, x2_s):
    n = pl.program_id(0)
    m = pl.program_id(1)

    x = x_ref[...]
    e = e_ref[...]

    @pl.when(m == 0)
    def _():
        et_ref[...] = e.T

    @pl.when(n == 0)
    def _():
        x2_s[m] = jnp.sum(x * x, axis=1)

    conv = jnp.dot((2.0 * x).astype(jnp.bfloat16), e,
                   preferred_element_type=jnp.float32)
    e2 = jnp.sum(e * e, axis=0)
    dist = (x2_s[m][:, None] - conv) + e2[None, :]

    lmin = jnp.min(dist, axis=1)
    iota = lax.broadcasted_iota(jnp.int32, (BM, BN), 1) + n * BN
    big = jnp.int32(2**31 - 1)
    lidx = jnp.min(jnp.where(dist == lmin[:, None], iota, big), axis=1)

    h = n // (NN // 2)

    @pl.when(n % (NN // 2) == 0)
    def _():
        minv_s[h, m] = lmin
        mini_s[h, m] = lidx

    @pl.when(n % (NN // 2) != 0)
    def _():
        better = lmin < minv_s[h, m]
        minv_s[h, m] = jnp.where(better, lmin, minv_s[h, m])
        mini_s[h, m] = jnp.where(better, lidx, mini_s[h, m])

    @pl.when(n == NN - 1)
    def _():
        v0 = minv_s[0, m]
        v0b = v0.astype(jnp.bfloat16).astype(jnp.float32)
        v1 = minv_s[1, m]
        take1 = v1 < v0b
        ind_ref[...] = jnp.where(take1, mini_s[1, m], mini_s[0, m])
        total = jnp.sum(jnp.where(take1, v1, v0))
        prev = jnp.where(m == 0, 0.0, dsum_ref[0, 0])
        s = prev + total
        dsum_ref[0, 0] = jnp.where(m == NM - 1, s / (M_TOKENS * DIM), s)


def _dist_argmin(x, embed):
    return pl.pallas_call(
        _dist_argmin_body,
        grid=(NN, NM),
        in_specs=[
            pl.BlockSpec((BM, DIM), lambda n, m: (m, 0)),
            pl.BlockSpec((DIM, BN), lambda n, m: (0, n)),
        ],
        out_specs=[
            pl.BlockSpec((BM,), lambda n, m: (m,)),
            pl.BlockSpec(memory_space=pltpu.SMEM),
            pl.BlockSpec((BN, DIM), lambda n, m: (n, 0)),
        ],
        out_shape=[
            jax.ShapeDtypeStruct((M_TOKENS,), jnp.int32),
            jax.ShapeDtypeStruct((1, 1), jnp.float32),
            jax.ShapeDtypeStruct((N_EMBED, DIM), jnp.float32),
        ],
        scratch_shapes=[
            pltpu.VMEM((2, NM, BM), jnp.float32),
            pltpu.VMEM((2, NM, BM), jnp.int32),
            pltpu.VMEM((NM, BM), jnp.float32),
        ],
    )(x, embed)


@functools.cache
def _make_sc_gather():
    info = plsc.get_sparse_core_info()
    nc, ns = info.num_cores, info.num_subcores
    nw = nc * ns
    chunks = M_TOKENS // nw // 128

    mesh = plsc.VectorSubcoreMesh(core_axis_name="c", subcore_axis_name="s")

    @functools.partial(
        pl.kernel,
        out_type=jax.ShapeDtypeStruct((nw, chunks, 128, DIM), jnp.float32),
        mesh=mesh,
        scratch_types=[
            pltpu.VMEM((chunks, 128), jnp.int32),
            pltpu.VMEM((chunks, 128, DIM), jnp.float32),
            pltpu.SemaphoreType.DMA,
        ],
    )
    def gather(table_hbm, idx_hbm, out_hbm, idx_v, rows_v, sem):
        wid = lax.axis_index("s") * nc + lax.axis_index("c")
        pltpu.sync_copy(idx_hbm.at[wid], idx_v)
        cps = [
            pltpu.async_copy(table_hbm.at[idx_v.at[j]], rows_v.at[j], sem)
            for j in range(chunks)
        ]
        for cp in cps:
            cp.wait()
        pltpu.sync_copy(rows_v, out_hbm.at[wid])

    return gather, nw, chunks


def kernel(input, embed):
    sc_gather, nw, chunks = _make_sc_gather()
    x = input.reshape(-1, DIM)
    ind_flat, dsum, embed_t = _dist_argmin(x, embed)
    idx_r = ind_flat.reshape(nw, chunks, 128)
    q = sc_gather(embed_t, idx_r)
    quantize = q.reshape(input.shape)
    diff = dsum[0, 0]
    embed_ind = ind_flat.reshape(input.shape[:-1])
    return (quantize, diff, embed_ind)


# speedup vs baseline: 1.0110x; 1.0110x over previous
import functools

import jax
import jax.numpy as jnp
from jax import lax
from jax.experimental import pallas as pl
from jax.experimental.pallas import tpu as pltpu
from jax.experimental.pallas import tpu_sc as plsc

DIM = 256
N_EMBED = 8192
M_TOKENS = 8192

BM = 1024
BN = 1024
NM = M_TOKENS // BM
NN = N_EMBED // BN


def _dist_argmin_body(x_ref, e_ref, ind_ref, dsum_ref, et_ref,
                      minv_s, mini_s, x2_s, e2_s):
    n = pl.program_id(0)
    m = pl.program_id(1)

    x = x_ref[...]
    e = e_ref[...]

    @pl.when(m == 0)
    def _():
        et_ref[...] = e.T
        e2_s[...] = jnp.sum(e * e, axis=0)

    @pl.when(n == 0)
    def _():
        x2_s[m] = jnp.sum(x * x, axis=1)

    conv = jnp.dot((2.0 * x).astype(jnp.bfloat16), e,
                   preferred_element_type=jnp.float32)
    dist = (x2_s[m][:, None] - conv) + e2_s[...][None, :]

    lmin = jnp.min(dist, axis=1)
    iota = lax.broadcasted_iota(jnp.int32, (BM, BN), 1)
    big = jnp.int32(2**31 - 1)
    lidx = jnp.min(jnp.where(dist == lmin[:, None], iota, big), axis=1) + n * BN

    h = n // (NN // 2)

    @pl.when(n % (NN // 2) == 0)
    def _():
        minv_s[h, m] = lmin
        mini_s[h, m] = lidx

    @pl.when(n % (NN // 2) != 0)
    def _():
        better = lmin < minv_s[h, m]
        minv_s[h, m] = jnp.where(better, lmin, minv_s[h, m])
        mini_s[h, m] = jnp.where(better, lidx, mini_s[h, m])

    @pl.when(n == NN - 1)
    def _():
        v0 = minv_s[0, m]
        v0b = v0.astype(jnp.bfloat16).astype(jnp.float32)
        v1 = minv_s[1, m]
        take1 = v1 < v0b
        ind_ref[...] = jnp.where(take1, mini_s[1, m], mini_s[0, m])
        total = jnp.sum(jnp.where(take1, v1, v0))
        prev = jnp.where(m == 0, 0.0, dsum_ref[0, 0])
        s = prev + total
        dsum_ref[0, 0] = jnp.where(m == NM - 1, s / (M_TOKENS * DIM), s)


def _dist_argmin(x, embed):
    return pl.pallas_call(
        _dist_argmin_body,
        grid=(NN, NM),
        in_specs=[
            pl.BlockSpec((BM, DIM), lambda n, m: (m, 0)),
            pl.BlockSpec((DIM, BN), lambda n, m: (0, n)),
        ],
        out_specs=[
            pl.BlockSpec((BM,), lambda n, m: (m,)),
            pl.BlockSpec(memory_space=pltpu.SMEM),
            pl.BlockSpec((BN, DIM), lambda n, m: (n, 0)),
        ],
        out_shape=[
            jax.ShapeDtypeStruct((M_TOKENS,), jnp.int32),
            jax.ShapeDtypeStruct((1, 1), jnp.float32),
            jax.ShapeDtypeStruct((N_EMBED, DIM), jnp.float32),
        ],
        scratch_shapes=[
            pltpu.VMEM((2, NM, BM), jnp.float32),
            pltpu.VMEM((2, NM, BM), jnp.int32),
            pltpu.VMEM((NM, BM), jnp.float32),
            pltpu.VMEM((BN,), jnp.float32),
        ],
    )(x, embed)


@functools.cache
def _make_sc_gather():
    info = plsc.get_sparse_core_info()
    nc, ns = info.num_cores, info.num_subcores
    nw = nc * ns
    chunks = M_TOKENS // nw // 128

    mesh = plsc.VectorSubcoreMesh(core_axis_name="c", subcore_axis_name="s")

    @functools.partial(
        pl.kernel,
        out_type=jax.ShapeDtypeStruct((nw, chunks, 128, DIM), jnp.float32),
        mesh=mesh,
        scratch_types=[
            pltpu.VMEM((chunks, 128), jnp.int32),
            pltpu.VMEM((chunks, 128, DIM), jnp.float32),
            pltpu.SemaphoreType.DMA,
        ],
    )
    def gather(table_hbm, idx_hbm, out_hbm, idx_v, rows_v, sem):
        wid = lax.axis_index("s") * nc + lax.axis_index("c")
        pltpu.sync_copy(idx_hbm.at[wid], idx_v)
        cps = [
            pltpu.async_copy(table_hbm.at[idx_v.at[j]], rows_v.at[j], sem)
            for j in range(chunks)
        ]
        for cp in cps:
            cp.wait()
        pltpu.sync_copy(rows_v, out_hbm.at[wid])

    return gather, nw, chunks


def kernel(input, embed):
    sc_gather, nw, chunks = _make_sc_gather()
    x = input.reshape(-1, DIM)
    ind_flat, dsum, embed_t = _dist_argmin(x, embed)
    idx_r = ind_flat.reshape(nw, chunks, 128)
    q = sc_gather(embed_t, idx_r)
    quantize = q.reshape(input.shape)
    diff = dsum[0, 0]
    embed_ind = ind_flat.reshape(input.shape[:-1])
    return (quantize, diff, embed_ind)


# speedup vs baseline: 1.4432x; 1.4275x over previous
import functools

import jax
import jax.numpy as jnp
from jax import lax
from jax.experimental import pallas as pl
from jax.experimental.pallas import tpu as pltpu
from jax.experimental.pallas import tpu_sc as plsc

DIM = 256
N_EMBED = 8192
M_TOKENS = 8192

BM = 1024
BN = 1024
NM = M_TOKENS // BM
NN = N_EMBED // BN


def _dist_argmin_body(x_ref, e_ref, ind_ref, dsum_ref, et_ref,
                      minv_s, mini_s, x2_s, e2_s):
    n = pl.program_id(0)
    m = pl.program_id(1)

    x = x_ref[...]
    e = e_ref[...]

    @pl.when(m == 0)
    def _():
        et_ref[...] = e.T
        e2_s[...] = jnp.sum(e * e, axis=0)[:, None]

    @pl.when(n == 0)
    def _():
        x2_s[m] = jnp.sum(x * x, axis=1)

    convT = lax.dot_general(e, (2.0 * x).astype(jnp.bfloat16),
                            (((0,), (1,)), ((), ())),
                            preferred_element_type=jnp.float32)
    distT = (x2_s[m][None, :] - convT) + e2_s[...]

    lmin = jnp.min(distT, axis=0)
    iota = lax.broadcasted_iota(jnp.int32, (BN, BM), 0)
    big = jnp.int32(2**31 - 1)
    lidx = jnp.min(jnp.where(distT == lmin[None, :], iota, big), axis=0) + n * BN

    h = n // (NN // 2)

    @pl.when(n % (NN // 2) == 0)
    def _():
        minv_s[h, m] = lmin
        mini_s[h, m] = lidx

    @pl.when(n % (NN // 2) != 0)
    def _():
        better = lmin < minv_s[h, m]
        minv_s[h, m] = jnp.where(better, lmin, minv_s[h, m])
        mini_s[h, m] = jnp.where(better, lidx, mini_s[h, m])

    @pl.when(n == NN - 1)
    def _():
        v0 = minv_s[0, m]
        v0b = v0.astype(jnp.bfloat16).astype(jnp.float32)
        v1 = minv_s[1, m]
        take1 = v1 < v0b
        ind_ref[...] = jnp.where(take1, mini_s[1, m], mini_s[0, m])
        total = jnp.sum(jnp.where(take1, v1, v0))
        prev = jnp.where(m == 0, 0.0, dsum_ref[0, 0])
        s = prev + total
        dsum_ref[0, 0] = jnp.where(m == NM - 1, s / (M_TOKENS * DIM), s)


def _dist_argmin(x, embed):
    return pl.pallas_call(
        _dist_argmin_body,
        grid=(NN, NM),
        in_specs=[
            pl.BlockSpec((BM, DIM), lambda n, m: (m, 0)),
            pl.BlockSpec((DIM, BN), lambda n, m: (0, n)),
        ],
        out_specs=[
            pl.BlockSpec((BM,), lambda n, m: (m,)),
            pl.BlockSpec(memory_space=pltpu.SMEM),
            pl.BlockSpec((BN, DIM), lambda n, m: (n, 0)),
        ],
        out_shape=[
            jax.ShapeDtypeStruct((M_TOKENS,), jnp.int32),
            jax.ShapeDtypeStruct((1, 1), jnp.float32),
            jax.ShapeDtypeStruct((N_EMBED, DIM), jnp.float32),
        ],
        scratch_shapes=[
            pltpu.VMEM((2, NM, BM), jnp.float32),
            pltpu.VMEM((2, NM, BM), jnp.int32),
            pltpu.VMEM((NM, BM), jnp.float32),
            pltpu.VMEM((BN, 1), jnp.float32),
        ],
    )(x, embed)


@functools.cache
def _make_sc_gather():
    info = plsc.get_sparse_core_info()
    nc, ns = info.num_cores, info.num_subcores
    nw = nc * ns
    chunks = M_TOKENS // nw // 128

    mesh = plsc.VectorSubcoreMesh(core_axis_name="c", subcore_axis_name="s")

    @functools.partial(
        pl.kernel,
        out_type=jax.ShapeDtypeStruct((nw, chunks, 128, DIM), jnp.float32),
        mesh=mesh,
        scratch_types=[
            pltpu.VMEM((chunks, 128), jnp.int32),
            pltpu.VMEM((chunks, 128, DIM), jnp.float32),
            pltpu.SemaphoreType.DMA,
        ],
    )
    def gather(table_hbm, idx_hbm, out_hbm, idx_v, rows_v, sem):
        wid = lax.axis_index("s") * nc + lax.axis_index("c")
        pltpu.sync_copy(idx_hbm.at[wid], idx_v)
        cps = [
            pltpu.async_copy(table_hbm.at[idx_v.at[j]], rows_v.at[j], sem)
            for j in range(chunks)
        ]
        for cp in cps:
            cp.wait()
        pltpu.sync_copy(rows_v, out_hbm.at[wid])

    return gather, nw, chunks


def kernel(input, embed):
    sc_gather, nw, chunks = _make_sc_gather()
    x = input.reshape(-1, DIM)
    ind_flat, dsum, embed_t = _dist_argmin(x, embed)
    idx_r = ind_flat.reshape(nw, chunks, 128)
    q = sc_gather(embed_t, idx_r)
    quantize = q.reshape(input.shape)
    diff = dsum[0, 0]
    embed_ind = ind_flat.reshape(input.shape[:-1])
    return (quantize, diff, embed_ind)


# speedup vs baseline: 1.6670x; 1.1551x over previous
import functools

import jax
import jax.numpy as jnp
from jax import lax
from jax.experimental import pallas as pl
from jax.experimental.pallas import tpu as pltpu
from jax.experimental.pallas import tpu_sc as plsc

DIM = 256
N_EMBED = 8192
M_TOKENS = 8192

BM = 1024
BN = 1024
NM = M_TOKENS // BM
NN = N_EMBED // BN


def _dist_argmin_body(x_ref, e_ref, ind_ref, dsum_ref, et_ref,
                      minv_s, mini_s, x2_s, e2_s):
    n = pl.program_id(0)
    m = pl.program_id(1)

    x = x_ref[...]
    e = e_ref[...]

    @pl.when(m == 0)
    def _():
        et_ref[...] = e.T
        e2_s[...] = jnp.sum(e * e, axis=0)[:, None]

    @pl.when(n == 0)
    def _():
        x2_s[m] = jnp.sum(x * x, axis=1)

    convT = lax.dot_general(e, (2.0 * x).astype(jnp.bfloat16),
                            (((0,), (1,)), ((), ())),
                            preferred_element_type=jnp.float32)
    distT = (x2_s[m][None, :] - convT) + e2_s[...]

    lmin = jnp.min(distT, axis=0)
    lidx = jnp.argmin(distT, axis=0).astype(jnp.int32) + n * BN

    h = n // (NN // 2)

    @pl.when(n % (NN // 2) == 0)
    def _():
        minv_s[h, m] = lmin
        mini_s[h, m] = lidx

    @pl.when(n % (NN // 2) != 0)
    def _():
        better = lmin < minv_s[h, m]
        minv_s[h, m] = jnp.where(better, lmin, minv_s[h, m])
        mini_s[h, m] = jnp.where(better, lidx, mini_s[h, m])

    @pl.when(n == NN - 1)
    def _():
        v0 = minv_s[0, m]
        v0b = v0.astype(jnp.bfloat16).astype(jnp.float32)
        v1 = minv_s[1, m]
        take1 = v1 < v0b
        ind_ref[...] = jnp.where(take1, mini_s[1, m], mini_s[0, m])
        total = jnp.sum(jnp.where(take1, v1, v0))
        prev = jnp.where(m == 0, 0.0, dsum_ref[0, 0])
        s = prev + total
        dsum_ref[0, 0] = jnp.where(m == NM - 1, s / (M_TOKENS * DIM), s)


def _dist_argmin(x, embed):
    return pl.pallas_call(
        _dist_argmin_body,
        grid=(NN, NM),
        in_specs=[
            pl.BlockSpec((BM, DIM), lambda n, m: (m, 0)),
            pl.BlockSpec((DIM, BN), lambda n, m: (0, n)),
        ],
        out_specs=[
            pl.BlockSpec((BM,), lambda n, m: (m,)),
            pl.BlockSpec(memory_space=pltpu.SMEM),
            pl.BlockSpec((BN, DIM), lambda n, m: (n, 0)),
        ],
        out_shape=[
            jax.ShapeDtypeStruct((M_TOKENS,), jnp.int32),
            jax.ShapeDtypeStruct((1, 1), jnp.float32),
            jax.ShapeDtypeStruct((N_EMBED, DIM), jnp.float32),
        ],
        scratch_shapes=[
            pltpu.VMEM((2, NM, BM), jnp.float32),
            pltpu.VMEM((2, NM, BM), jnp.int32),
            pltpu.VMEM((NM, BM), jnp.float32),
            pltpu.VMEM((BN, 1), jnp.float32),
        ],
    )(x, embed)


@functools.cache
def _make_sc_gather():
    info = plsc.get_sparse_core_info()
    nc, ns = info.num_cores, info.num_subcores
    nw = nc * ns
    chunks = M_TOKENS // nw // 128

    mesh = plsc.VectorSubcoreMesh(core_axis_name="c", subcore_axis_name="s")

    @functools.partial(
        pl.kernel,
        out_type=jax.ShapeDtypeStruct((nw, chunks, 128, DIM), jnp.float32),
        mesh=mesh,
        scratch_types=[
            pltpu.VMEM((chunks, 128), jnp.int32),
            pltpu.VMEM((chunks, 128, DIM), jnp.float32),
            pltpu.SemaphoreType.DMA,
        ],
    )
    def gather(table_hbm, idx_hbm, out_hbm, idx_v, rows_v, sem):
        wid = lax.axis_index("s") * nc + lax.axis_index("c")
        pltpu.sync_copy(idx_hbm.at[wid], idx_v)
        cps = [
            pltpu.async_copy(table_hbm.at[idx_v.at[j]], rows_v.at[j], sem)
            for j in range(chunks)
        ]
        for cp in cps:
            cp.wait()
        pltpu.sync_copy(rows_v, out_hbm.at[wid])

    return gather, nw, chunks


def kernel(input, embed):
    sc_gather, nw, chunks = _make_sc_gather()
    x = input.reshape(-1, DIM)
    ind_flat, dsum, embed_t = _dist_argmin(x, embed)
    idx_r = ind_flat.reshape(nw, chunks, 128)
    q = sc_gather(embed_t, idx_r)
    quantize = q.reshape(input.shape)
    diff = dsum[0, 0]
    embed_ind = ind_flat.reshape(input.shape[:-1])
    return (quantize, diff, embed_ind)


# speedup vs baseline: 1.7841x; 1.0702x over previous
import functools

import jax
import jax.numpy as jnp
from jax import lax
from jax.experimental import pallas as pl
from jax.experimental.pallas import tpu as pltpu
from jax.experimental.pallas import tpu_sc as plsc

DIM = 256
N_EMBED = 8192
M_TOKENS = 8192

BM = 1024
BN = 2048
NM = M_TOKENS // BM
NN = N_EMBED // BN


def _dist_argmin_body(x_ref, e_ref, ind_ref, dsum_ref, et_ref,
                      minv_s, mini_s, x2_s, e2_s):
    n = pl.program_id(0)
    m = pl.program_id(1)

    x = x_ref[...]
    e = e_ref[...]

    @pl.when(m == 0)
    def _():
        et_ref[...] = e.T
        e2_s[...] = jnp.sum(e * e, axis=0)[:, None]

    @pl.when(n == 0)
    def _():
        x2_s[m] = jnp.sum(x * x, axis=1)

    convT = lax.dot_general(e, (2.0 * x).astype(jnp.bfloat16),
                            (((0,), (1,)), ((), ())),
                            preferred_element_type=jnp.float32)
    distT = (x2_s[m][None, :] - convT) + e2_s[...]

    lmin = jnp.min(distT, axis=0)
    lidx = jnp.argmin(distT, axis=0).astype(jnp.int32) + n * BN

    h = n // (NN // 2)

    @pl.when(n % (NN // 2) == 0)
    def _():
        minv_s[h, m] = lmin
        mini_s[h, m] = lidx

    @pl.when(n % (NN // 2) != 0)
    def _():
        better = lmin < minv_s[h, m]
        minv_s[h, m] = jnp.where(better, lmin, minv_s[h, m])
        mini_s[h, m] = jnp.where(better, lidx, mini_s[h, m])

    @pl.when(n == NN - 1)
    def _():
        v0 = minv_s[0, m]
        v0b = v0.astype(jnp.bfloat16).astype(jnp.float32)
        v1 = minv_s[1, m]
        take1 = v1 < v0b
        ind_ref[...] = jnp.where(take1, mini_s[1, m], mini_s[0, m])
        total = jnp.sum(jnp.where(take1, v1, v0))
        prev = jnp.where(m == 0, 0.0, dsum_ref[0, 0])
        s = prev + total
        dsum_ref[0, 0] = jnp.where(m == NM - 1, s / (M_TOKENS * DIM), s)


def _dist_argmin(x, embed):
    return pl.pallas_call(
        _dist_argmin_body,
        grid=(NN, NM),
        in_specs=[
            pl.BlockSpec((BM, DIM), lambda n, m: (m, 0)),
            pl.BlockSpec((DIM, BN), lambda n, m: (0, n)),
        ],
        out_specs=[
            pl.BlockSpec((BM,), lambda n, m: (m,)),
            pl.BlockSpec(memory_space=pltpu.SMEM),
            pl.BlockSpec((BN, DIM), lambda n, m: (n, 0)),
        ],
        out_shape=[
            jax.ShapeDtypeStruct((M_TOKENS,), jnp.int32),
            jax.ShapeDtypeStruct((1, 1), jnp.float32),
            jax.ShapeDtypeStruct((N_EMBED, DIM), jnp.float32),
        ],
        scratch_shapes=[
            pltpu.VMEM((2, NM, BM), jnp.float32),
            pltpu.VMEM((2, NM, BM), jnp.int32),
            pltpu.VMEM((NM, BM), jnp.float32),
            pltpu.VMEM((BN, 1), jnp.float32),
        ],
    )(x, embed)


@functools.cache
def _make_sc_gather():
    info = plsc.get_sparse_core_info()
    nc, ns = info.num_cores, info.num_subcores
    nw = nc * ns
    chunks = M_TOKENS // nw // 128

    mesh = plsc.VectorSubcoreMesh(core_axis_name="c", subcore_axis_name="s")

    @functools.partial(
        pl.kernel,
        out_type=jax.ShapeDtypeStruct((nw, chunks, 128, DIM), jnp.float32),
        mesh=mesh,
        scratch_types=[
            pltpu.VMEM((chunks, 128), jnp.int32),
            pltpu.VMEM((chunks, 128, DIM), jnp.float32),
            pltpu.SemaphoreType.DMA,
        ],
    )
    def gather(table_hbm, idx_hbm, out_hbm, idx_v, rows_v, sem):
        wid = lax.axis_index("s") * nc + lax.axis_index("c")
        pltpu.sync_copy(idx_hbm.at[wid], idx_v)
        cps = [
            pltpu.async_copy(table_hbm.at[idx_v.at[j]], rows_v.at[j], sem)
            for j in range(chunks)
        ]
        for cp in cps:
            cp.wait()
        pltpu.sync_copy(rows_v, out_hbm.at[wid])

    return gather, nw, chunks


def kernel(input, embed):
    sc_gather, nw, chunks = _make_sc_gather()
    x = input.reshape(-1, DIM)
    ind_flat, dsum, embed_t = _dist_argmin(x, embed)
    idx_r = ind_flat.reshape(nw, chunks, 128)
    q = sc_gather(embed_t, idx_r)
    quantize = q.reshape(input.shape)
    diff = dsum[0, 0]
    embed_ind = ind_flat.reshape(input.shape[:-1])
    return (quantize, diff, embed_ind)
